# baseline (device time: 96080 ns/iter reference)
import functools

import jax
import jax.numpy as jnp
from jax import lax
from jax.experimental import pallas as pl
from jax.experimental.pallas import tpu as pltpu

N_DEV = 8


def kernel(x, router_W, route_idx, expert_W, shared_W):
    n_per, d = x.shape
    n_exp_local, _, h = expert_W.shape
    n_exp = router_W.shape[1]

    def body(x_ref, rw_ref, idx_ref, ew_ref, sw_ref, out_ref,
             xall, idxall, acc, accb, comm,
             xsend, xrecv, isend, irecv, rs_send, rs_recv):
        my = lax.axis_index("i")
        left = jnp.mod(my - 1, N_DEV)
        right = jnp.mod(my + 1, N_DEV)

        barrier = pltpu.get_barrier_semaphore()
        for nbr in (left, right):
            pl.semaphore_signal(barrier, inc=1, device_id=(nbr,),
                                device_id_type=pl.DeviceIdType.MESH)
        pl.semaphore_wait(barrier, 2)

        xall[my] = x_ref[...].astype(jnp.bfloat16)
        idxall[my] = idx_ref[...]

        rw = rw_ref[...]
        ewb = ew_ref[...].astype(jnp.bfloat16)
        wcat = jnp.concatenate([ewb[j] for j in range(n_exp_local)], axis=1)
        e0 = my * n_exp_local
        cols = lax.broadcasted_iota(jnp.int32, (n_per, n_exp), 1)

        def compute_chunk(c):
            xc = xall[c]
            idxc = idxall[c]
            scores = jnp.dot(xc.astype(jnp.float32), rw,
                             preferred_element_type=jnp.float32)
            smax = jnp.max(scores, axis=-1, keepdims=True)
            ex = jnp.exp(scores - smax)
            denom = jnp.sum(ex, axis=-1, keepdims=True)
            pe = jnp.sum(jnp.where(cols == idxc, ex, 0.0), axis=-1,
                         keepdims=True) / denom
            y = jnp.dot(xc, wcat, preferred_element_type=jnp.float32)
            accv = jnp.zeros((n_per, h), jnp.float32)
            for j in range(n_exp_local):
                coef = jnp.where(idxc == e0 + j, pe, 0.0)
                accv = accv + coef * y[:, j * h:(j + 1) * h]
            acc[c] = accv
            accb[c] = accv.astype(jnp.bfloat16)

        ag = []
        for k in range(N_DEV - 1):
            o = jnp.mod(my - k, N_DEV)
            ag.append((
                pltpu.make_async_remote_copy(
                    src_ref=xall.at[o], dst_ref=xall.at[o],
                    send_sem=xsend.at[k], recv_sem=xrecv.at[k],
                    device_id=(right,), device_id_type=pl.DeviceIdType.MESH),
                pltpu.make_async_remote_copy(
                    src_ref=idxall.at[o], dst_ref=idxall.at[o],
                    send_sem=isend.at[k], recv_sem=irecv.at[k],
                    device_id=(right,), device_id_type=pl.DeviceIdType.MESH),
            ))
        rs = []
        for s in range(N_DEV - 1):
            c_s = jnp.mod(my - s - 1, N_DEV)
            rs.append(pltpu.make_async_remote_copy(
                src_ref=accb.at[c_s], dst_ref=comm.at[s],
                send_sem=rs_send.at[s], recv_sem=rs_recv.at[s],
                device_id=(right,), device_id_type=pl.DeviceIdType.MESH))

        ag[0][0].start()
        ag[0][1].start()
        compute_chunk(my)
        for s in range(N_DEV - 1):
            ag[s][0].wait_recv()
            ag[s][1].wait_recv()
            if s + 1 < N_DEV - 1:
                ag[s + 1][0].start()
                ag[s + 1][1].start()
            c = jnp.mod(my - s - 1, N_DEV)
            compute_chunk(c)
            if s > 0:
                rs[s - 1].wait_recv()
                accv = acc[c] + comm[s - 1].astype(jnp.float32)
                acc[c] = accv
                accb[c] = accv.astype(jnp.bfloat16)
            rs[s].start()
        rs[N_DEV - 2].wait_recv()
        acc[my] = acc[my] + comm[N_DEV - 2].astype(jnp.float32)

        shared = jnp.dot(x_ref[...], sw_ref[...],
                         preferred_element_type=jnp.float32)
        out_ref[...] = acc[my] + shared

        for s in range(N_DEV - 1):
            ag[s][0].wait_send()
            ag[s][1].wait_send()
            rs[s].wait_send()

        @functools.partial(pl.run_scoped,
                           second_barrier=pltpu.SemaphoreType.REGULAR)
        def _(second_barrier):
            for nbr in (left, right):
                pl.semaphore_signal(second_barrier, inc=1, device_id=(nbr,),
                                    device_id_type=pl.DeviceIdType.MESH)
            pl.semaphore_wait(second_barrier, 2)

    return pl.pallas_call(
        body,
        out_shape=jax.ShapeDtypeStruct((n_per, h), jnp.float32),
        in_specs=[pl.BlockSpec(memory_space=pltpu.VMEM)] * 5,
        out_specs=pl.BlockSpec(memory_space=pltpu.VMEM),
        scratch_shapes=[
            pltpu.VMEM((N_DEV, n_per, d), jnp.bfloat16),
            pltpu.VMEM((N_DEV, n_per, 1), jnp.int32),
            pltpu.VMEM((N_DEV, n_per, h), jnp.float32),
            pltpu.VMEM((N_DEV, n_per, h), jnp.bfloat16),
            pltpu.VMEM((N_DEV - 1, n_per, h), jnp.bfloat16),
            pltpu.SemaphoreType.DMA((N_DEV - 1,)),
            pltpu.SemaphoreType.DMA((N_DEV - 1,)),
            pltpu.SemaphoreType.DMA((N_DEV - 1,)),
            pltpu.SemaphoreType.DMA((N_DEV - 1,)),
            pltpu.SemaphoreType.DMA((N_DEV - 1,)),
            pltpu.SemaphoreType.DMA((N_DEV - 1,)),
        ],
        compiler_params=pltpu.CompilerParams(collective_id=0),
    )(x, router_W, route_idx, expert_W, shared_W)


# device time: 55472 ns/iter; 1.7320x vs baseline; 1.7320x over previous
import functools

import jax
import jax.numpy as jnp
from jax import lax
from jax.experimental import pallas as pl
from jax.experimental.pallas import tpu as pltpu

N_DEV = 8


def kernel(x, router_W, route_idx, expert_W, shared_W):
    n_per, d = x.shape
    n_exp_local, _, h = expert_W.shape
    n_exp = router_W.shape[1]
    n_half = n_per // 2

    def body(x_ref, rw_ref, idx_ref, ew_ref, sw_ref, out_ref,
             xaR, xaL, ohR, ohL, accR, accL, abR, abL, cmR, cmL,
             xsR, xrR, osR, orR, rsR_s, rsR_r,
             xsL, xrL, osL, orL, rsL_s, rsL_r):
        my = lax.axis_index("i")
        left = jnp.mod(my - 1, N_DEV)
        right = jnp.mod(my + 1, N_DEV)

        barrier = pltpu.get_barrier_semaphore()
        for nbr in (left, right):
            pl.semaphore_signal(barrier, inc=1, device_id=(nbr,),
                                device_id_type=pl.DeviceIdType.MESH)
        pl.semaphore_wait(barrier, 2)

        xf = x_ref[...]
        rw = rw_ref[...]
        idxc = idx_ref[...]
        cols = lax.broadcasted_iota(jnp.int32, (n_per, n_exp), 1)
        scores = jnp.dot(xf, rw, preferred_element_type=jnp.float32)
        smax = jnp.max(scores, axis=-1, keepdims=True)
        ex = jnp.exp(scores - smax)
        pe = ex / jnp.sum(ex, axis=-1, keepdims=True)
        oh = jnp.where(cols == idxc, pe, 0.0).astype(jnp.bfloat16)

        xb = xf.astype(jnp.bfloat16)
        xaR[my] = xb[:n_half]
        xaL[my] = xb[n_half:]
        ohR[my] = oh[:n_half]
        ohL[my] = oh[n_half:]

        ewb = ew_ref[...].astype(jnp.bfloat16)
        wcat = jnp.concatenate([ewb[j] for j in range(n_exp_local)], axis=1)
        e0 = my * n_exp_local
        hcols = lax.broadcasted_iota(jnp.int32, (n_half, n_exp), 1)

        def compute_half(xa, ohh, acc, accb, c):
            xc = xa[c]
            ohc = ohh[c].astype(jnp.float32)
            y = jnp.dot(xc, wcat, preferred_element_type=jnp.float32)
            accv = jnp.zeros((n_half, h), jnp.float32)
            for j in range(n_exp_local):
                coef = jnp.sum(jnp.where(hcols == e0 + j, ohc, 0.0),
                               axis=-1, keepdims=True)
                accv = accv + coef * y[:, j * h:(j + 1) * h]
            acc[c] = accv
            accb[c] = accv.astype(jnp.bfloat16)

        def mk(src, dst, ssem, rsem, dev):
            return pltpu.make_async_remote_copy(
                src_ref=src, dst_ref=dst, send_sem=ssem, recv_sem=rsem,
                device_id=(dev,), device_id_type=pl.DeviceIdType.MESH)

        agR, agL, rsR, rsL = [], [], [], []
        for k in range(N_DEV - 1):
            oR = jnp.mod(my - k, N_DEV)
            oL = jnp.mod(my + k, N_DEV)
            agR.append((mk(xaR.at[oR], xaR.at[oR], xsR.at[k], xrR.at[k], right),
                        mk(ohR.at[oR], ohR.at[oR], osR.at[k], orR.at[k], right)))
            agL.append((mk(xaL.at[oL], xaL.at[oL], xsL.at[k], xrL.at[k], left),
                        mk(ohL.at[oL], ohL.at[oL], osL.at[k], orL.at[k], left)))
            cR = jnp.mod(my - k - 1, N_DEV)
            cL = jnp.mod(my + k + 1, N_DEV)
            rsR.append(mk(abR.at[cR], cmR.at[k], rsR_s.at[k], rsR_r.at[k], right))
            rsL.append(mk(abL.at[cL], cmL.at[k], rsL_s.at[k], rsL_r.at[k], left))

        for dsc in (*agR[0], *agL[0]):
            dsc.start()
        compute_half(xaR, ohR, accR, abR, my)
        compute_half(xaL, ohL, accL, abL, my)
        for s in range(N_DEV - 1):
            for dsc in agR[s]:
                dsc.wait_recv()
            if s + 1 < N_DEV - 1:
                for dsc in agR[s + 1]:
                    dsc.start()
            cR = jnp.mod(my - s - 1, N_DEV)
            compute_half(xaR, ohR, accR, abR, cR)
            if s > 0:
                rsR[s - 1].wait_recv()
                av = accR[cR] + cmR[s - 1].astype(jnp.float32)
                accR[cR] = av
                abR[cR] = av.astype(jnp.bfloat16)
            rsR[s].start()
            for dsc in agL[s]:
                dsc.wait_recv()
            if s + 1 < N_DEV - 1:
                for dsc in agL[s + 1]:
                    dsc.start()
            cL = jnp.mod(my + s + 1, N_DEV)
            compute_half(xaL, ohL, accL, abL, cL)
            if s > 0:
                rsL[s - 1].wait_recv()
                av = accL[cL] + cmL[s - 1].astype(jnp.float32)
                accL[cL] = av
                abL[cL] = av.astype(jnp.bfloat16)
            rsL[s].start()
        rsR[N_DEV - 2].wait_recv()
        rsL[N_DEV - 2].wait_recv()

        sw = sw_ref[...]
        out_ref[:n_half] = (
            accR[my] + cmR[N_DEV - 2].astype(jnp.float32)
            + jnp.dot(xf[:n_half], sw, preferred_element_type=jnp.float32))
        out_ref[n_half:] = (
            accL[my] + cmL[N_DEV - 2].astype(jnp.float32)
            + jnp.dot(xf[n_half:], sw, preferred_element_type=jnp.float32))

        for s in range(N_DEV - 1):
            for dsc in (*agR[s], *agL[s], rsR[s], rsL[s]):
                dsc.wait_send()

        @functools.partial(pl.run_scoped,
                           second_barrier=pltpu.SemaphoreType.REGULAR)
        def _(second_barrier):
            for nbr in (left, right):
                pl.semaphore_signal(second_barrier, inc=1, device_id=(nbr,),
                                    device_id_type=pl.DeviceIdType.MESH)
            pl.semaphore_wait(second_barrier, 2)

    dma7 = pltpu.SemaphoreType.DMA((N_DEV - 1,))
    return pl.pallas_call(
        body,
        out_shape=jax.ShapeDtypeStruct((n_per, h), jnp.float32),
        in_specs=[pl.BlockSpec(memory_space=pltpu.VMEM)] * 5,
        out_specs=pl.BlockSpec(memory_space=pltpu.VMEM),
        scratch_shapes=[
            pltpu.VMEM((N_DEV, n_half, d), jnp.bfloat16),
            pltpu.VMEM((N_DEV, n_half, d), jnp.bfloat16),
            pltpu.VMEM((N_DEV, n_half, n_exp), jnp.bfloat16),
            pltpu.VMEM((N_DEV, n_half, n_exp), jnp.bfloat16),
            pltpu.VMEM((N_DEV, n_half, h), jnp.float32),
            pltpu.VMEM((N_DEV, n_half, h), jnp.float32),
            pltpu.VMEM((N_DEV, n_half, h), jnp.bfloat16),
            pltpu.VMEM((N_DEV, n_half, h), jnp.bfloat16),
            pltpu.VMEM((N_DEV - 1, n_half, h), jnp.bfloat16),
            pltpu.VMEM((N_DEV - 1, n_half, h), jnp.bfloat16),
            dma7, dma7, dma7, dma7, dma7, dma7,
            dma7, dma7, dma7, dma7, dma7, dma7,
        ],
        compiler_params=pltpu.CompilerParams(collective_id=0),
    )(x, router_W, route_idx, expert_W, shared_W)
